# Initial kernel scaffold; baseline (speedup 1.0000x reference)
import jax
import jax.numpy as jnp
from jax.experimental import pallas as pl


def kernel(x, ffm_emb, lin_emb, lin_bias, W1, b1, W2, b2, W3, b3):
    raise NotImplementedError("write your pallas kernel here")


# speedup vs baseline: 2.3345x; 2.3345x over previous
import functools

import jax
import jax.numpy as jnp
import numpy as np
from jax import lax
from jax.experimental import pallas as pl
from jax.experimental.pallas import tpu as pltpu
from jax.experimental.pallas import tpu_sc as plsc

F = 26
D = 16
FD = 3846
V = F * FD
B = 4096
P = F * (F - 1) // 2
PPAD = 328
CROSSP = PPAD * D
EPS = 1e-5
S = 1.0 / float(np.sqrt(1.0 + EPS))

NC, NS = 2, 16
NW = NC * NS
BCHUNK = 512
NCHUNK = B // BCHUNK
SUBR = BCHUNK // 128
PGRPS = NW // NCHUNK
PB = PPAD // 8
KMAX = (PB + PGRPS - 1) // PGRPS
ROWS_PER_W = B // NW

_IU, _JU = np.triu_indices(F, k=1)
_OFF = (np.arange(F, dtype=np.int64) * FD).astype(np.int32)


@functools.cache
def _make_sc_gather_cross():
  mesh = plsc.VectorSubcoreMesh(core_axis_name="c", subcore_axis_name="s",
                                num_cores=NC, num_subcores=NS)

  @functools.partial(
      pl.kernel,
      out_type=[
          jax.ShapeDtypeStruct((B, CROSSP), jnp.float32),
          jax.ShapeDtypeStruct((B,), jnp.float32),
      ],
      mesh=mesh,
      compiler_params=pltpu.CompilerParams(use_tc_tiling_on_sc=False),
      scratch_types=[
          pltpu.VMEM((8, SUBR, 128), jnp.int32),
          pltpu.VMEM((8, SUBR, 128), jnp.int32),
          pltpu.VMEM((BCHUNK, D), jnp.float32),
          pltpu.VMEM((BCHUNK, D), jnp.float32),
          pltpu.VMEM((BCHUNK, 8 * D), jnp.float32),
          pltpu.VMEM((ROWS_PER_W,), jnp.int32),
          pltpu.VMEM((ROWS_PER_W,), jnp.float32),
          pltpu.VMEM((ROWS_PER_W,), jnp.float32),
          pltpu.SemaphoreType.DMA,
          pltpu.SemaphoreType.DMA,
          pltpu.SemaphoreType.DMA,
      ],
  )
  def _sc_gather_cross(ffm, idxa, idxb, lidx_hbm, lin_tab, cross_out, lin_out,
                       idxa_v, idxb_v, rowsa_v, rowsb_v, prod_v,
                       lidx_v, lrow_v, lsum_v, sema, semb, seml):
      wid = lax.axis_index("s") * NC + lax.axis_index("c")

      cchunk = lax.rem(wid, NCHUNK)
      pgrp = lax.div(wid, NCHUNK)
      row0 = cchunk * BCHUNK

      def pair_step(k, carry):
          pb = pgrp + PGRPS * k

          @pl.when(pb < PB)
          def _():
              pltpu.sync_copy(idxa.at[cchunk, pl.ds(pb * 8, 8), :, :], idxa_v)
              pltpu.sync_copy(idxb.at[cchunk, pl.ds(pb * 8, 8), :, :], idxb_v)
              for q in range(8):
                  waits = []
                  for h in range(SUBR):
                      waits.append(pltpu.async_copy(
                          ffm.at[idxa_v.at[q, h]],
                          rowsa_v.at[pl.ds(h * 128, 128), :], sema))
                      waits.append(pltpu.async_copy(
                          ffm.at[idxb_v.at[q, h]],
                          rowsb_v.at[pl.ds(h * 128, 128), :], semb))
                  for w in waits:
                      w.wait()

                  @plsc.parallel_loop(0, BCHUNK, unroll=8)
                  def _mul(r):
                      prod_v[r, pl.ds(q * D, D)] = rowsa_v[r, :] * rowsb_v[r, :]

              pltpu.sync_copy(
                  prod_v,
                  cross_out.at[pl.ds(row0, BCHUNK), pl.ds(pb * 128, 128)])

          return carry

      lax.fori_loop(0, KMAX, pair_step, 0)

      lbase = wid * ROWS_PER_W
      for k8 in range(8):
          lsum_v[pl.ds(k8 * 16, 16)] = jnp.zeros((16,), jnp.float32)

      def lin_step(g, carry):
          pltpu.sync_copy(lidx_hbm.at[g, wid, :], lidx_v)
          pltpu.async_copy(lin_tab.at[lidx_v], lrow_v, seml).wait()
          for k8 in range(8):
              sl = pl.ds(k8 * 16, 16)
              lsum_v[sl] = lsum_v[sl] + lrow_v[sl]
          return carry

      lax.fori_loop(0, F, lin_step, 0)
      pltpu.sync_copy(lsum_v, lin_out.at[pl.ds(lbase, ROWS_PER_W)])

  return _sc_gather_cross


def _tc_mlp_body(cross_ref, lin_ref, w1_ref, b1_ref, w2_ref, b2_ref, w3_ref,
                 cb_ref, out_ref):
    t = jnp.dot(cross_ref[...], w1_ref[...], preferred_element_type=jnp.float32)
    h1 = jnp.maximum(t * (S * S) + b1_ref[...] * S, 0.0)
    h2 = jnp.dot(h1, w2_ref[...], preferred_element_type=jnp.float32) * S \
        + b2_ref[...] * S
    h2 = jnp.maximum(h2, 0.0)
    o3 = jnp.sum(h2 * w3_ref[...], axis=1)
    out_ref[...] = jax.nn.sigmoid(lin_ref[...] + o3 + cb_ref[0])


_BM = 512
_tc_mlp = pl.pallas_call(
    _tc_mlp_body,
    grid=(B // _BM,),
    in_specs=[
        pl.BlockSpec((_BM, CROSSP), lambda i: (i, 0)),
        pl.BlockSpec((_BM,), lambda i: (i,)),
        pl.BlockSpec((CROSSP, 64), lambda i: (0, 0)),
        pl.BlockSpec((1, 64), lambda i: (0, 0)),
        pl.BlockSpec((64, 32), lambda i: (0, 0)),
        pl.BlockSpec((1, 32), lambda i: (0, 0)),
        pl.BlockSpec((1, 32), lambda i: (0, 0)),
        pl.BlockSpec(memory_space=pltpu.SMEM),
    ],
    out_specs=pl.BlockSpec((_BM,), lambda i: (i,)),
    out_shape=jax.ShapeDtypeStruct((B,), jnp.float32),
)


def kernel(x, ffm_emb, lin_emb, lin_bias, W1, b1, W2, b2, W3, b3):
    x32 = x.astype(jnp.int32)
    xo = x32 + jnp.asarray(_OFF)[None, :]
    iu = np.concatenate([_IU, _IU[:PPAD - P]])
    ju = np.concatenate([_JU, _JU[:PPAD - P]])
    ja = jnp.asarray((ju.astype(np.int64) * V).astype(np.int32))
    ib = jnp.asarray((iu.astype(np.int64) * V).astype(np.int32))
    idxa = (xo[:, iu] + ja[None, :]).T.reshape(PPAD, NCHUNK, SUBR, 128)
    idxa = idxa.transpose(1, 0, 2, 3)
    idxb = (xo[:, ju] + ib[None, :]).T.reshape(PPAD, NCHUNK, SUBR, 128)
    idxb = idxb.transpose(1, 0, 2, 3)
    lidx = xo.T.reshape(F, NW, ROWS_PER_W)
    ffm_flat = ffm_emb.reshape(F * V, D)
    lin_flat = lin_emb.reshape(V)

    cross, lin = _make_sc_gather_cross()(ffm_flat, idxa, idxb, lidx, lin_flat)

    w1p = jnp.pad(W1, ((0, CROSSP - P * D), (0, 0)))
    cb = (lin_bias + b3).reshape(1).astype(jnp.float32)
    return _tc_mlp(cross, lin, w1p, b1.reshape(1, 64), W2, b2.reshape(1, 32),
                   W3.reshape(1, 32), cb)


# speedup vs baseline: 2.3469x; 1.0053x over previous
import functools

import jax
import jax.numpy as jnp
import numpy as np
from jax import lax
from jax.experimental import pallas as pl
from jax.experimental.pallas import tpu as pltpu
from jax.experimental.pallas import tpu_sc as plsc

F = 26
D = 16
FD = 3846
V = F * FD
B = 4096
P = F * (F - 1) // 2
PPAD = 328
CROSSP = PPAD * D
EPS = 1e-5
S = 1.0 / float(np.sqrt(1.0 + EPS))

NC, NS = 2, 16
NW = NC * NS
BCHUNK = 512
NCHUNK = B // BCHUNK
SUBR = BCHUNK // 128
PGRPS = NW // NCHUNK
PB = PPAD // 8
KMAX = (PB + PGRPS - 1) // PGRPS
ROWS_PER_W = B // NW

_IU, _JU = np.triu_indices(F, k=1)
_OFF = (np.arange(F, dtype=np.int64) * FD).astype(np.int32)


@functools.cache
def _make_sc_gather_cross():
  mesh = plsc.VectorSubcoreMesh(core_axis_name="c", subcore_axis_name="s",
                                num_cores=NC, num_subcores=NS)

  @functools.partial(
      pl.kernel,
      out_type=[
          jax.ShapeDtypeStruct((B, CROSSP), jnp.float32),
          jax.ShapeDtypeStruct((B,), jnp.float32),
      ],
      mesh=mesh,
      compiler_params=pltpu.CompilerParams(use_tc_tiling_on_sc=False),
      scratch_types=[
          pltpu.VMEM((8, SUBR, 128), jnp.int32),
          pltpu.VMEM((8, SUBR, 128), jnp.int32),
          pltpu.VMEM((BCHUNK, D), jnp.float32),
          pltpu.VMEM((BCHUNK, D), jnp.float32),
          pltpu.VMEM((BCHUNK, 8 * D), jnp.float32),
          pltpu.VMEM((ROWS_PER_W,), jnp.int32),
          pltpu.VMEM((ROWS_PER_W,), jnp.float32),
          pltpu.VMEM((ROWS_PER_W,), jnp.float32),
          pltpu.SemaphoreType.DMA,
          pltpu.SemaphoreType.DMA,
          pltpu.SemaphoreType.DMA,
      ],
  )
  def _sc_gather_cross(ffm, idxa, idxb, lidx_hbm, lin_tab, cross_out, lin_out,
                       idxa_v, idxb_v, rowsa_v, rowsb_v, prod_v,
                       lidx_v, lrow_v, lsum_v, sema, semb, seml):
      wid = lax.axis_index("s") * NC + lax.axis_index("c")

      cchunk = lax.rem(wid, NCHUNK)
      pgrp = lax.div(wid, NCHUNK)
      row0 = cchunk * BCHUNK

      def pair_step(k, carry):
          pb = pgrp + PGRPS * k

          @pl.when(pb < PB)
          def _():
              pltpu.sync_copy(
                  idxa.at[pl.ds(pb * 8, 8), pl.ds(cchunk * SUBR, SUBR), :],
                  idxa_v)
              pltpu.sync_copy(
                  idxb.at[pl.ds(pb * 8, 8), pl.ds(cchunk * SUBR, SUBR), :],
                  idxb_v)
              for q in range(8):
                  waits = []
                  for h in range(SUBR):
                      waits.append(pltpu.async_copy(
                          ffm.at[idxa_v.at[q, h]],
                          rowsa_v.at[pl.ds(h * 128, 128), :], sema))
                      waits.append(pltpu.async_copy(
                          ffm.at[idxb_v.at[q, h]],
                          rowsb_v.at[pl.ds(h * 128, 128), :], semb))
                  for w in waits:
                      w.wait()

                  @plsc.parallel_loop(0, BCHUNK, unroll=8)
                  def _mul(r):
                      prod_v[r, pl.ds(q * D, D)] = rowsa_v[r, :] * rowsb_v[r, :]

              pltpu.sync_copy(
                  prod_v,
                  cross_out.at[pl.ds(row0, BCHUNK), pl.ds(pb * 128, 128)])

          return carry

      lax.fori_loop(0, KMAX, pair_step, 0)

      lbase = wid * ROWS_PER_W
      for k8 in range(8):
          lsum_v[pl.ds(k8 * 16, 16)] = jnp.zeros((16,), jnp.float32)

      def lin_step(g, carry):
          pltpu.sync_copy(lidx_hbm.at[g, wid, :], lidx_v)
          pltpu.async_copy(lin_tab.at[lidx_v], lrow_v, seml).wait()
          for k8 in range(8):
              sl = pl.ds(k8 * 16, 16)
              lsum_v[sl] = lsum_v[sl] + lrow_v[sl]
          return carry

      lax.fori_loop(0, F, lin_step, 0)
      pltpu.sync_copy(lsum_v, lin_out.at[pl.ds(lbase, ROWS_PER_W)])

  return _sc_gather_cross


def _tc_mlp_body(cross_ref, lin_ref, w1_ref, b1_ref, w2_ref, b2_ref, w3_ref,
                 cb_ref, out_ref):
    t = jnp.dot(cross_ref[...], w1_ref[...], preferred_element_type=jnp.float32)
    h1 = jnp.maximum(t * (S * S) + b1_ref[...] * S, 0.0)
    h2 = jnp.dot(h1, w2_ref[...], preferred_element_type=jnp.float32) * S \
        + b2_ref[...] * S
    h2 = jnp.maximum(h2, 0.0)
    o3 = jnp.sum(h2 * w3_ref[...], axis=1)
    out_ref[...] = jax.nn.sigmoid(lin_ref[...] + o3 + cb_ref[0])


_BM = 512
_tc_mlp = pl.pallas_call(
    _tc_mlp_body,
    grid=(B // _BM,),
    in_specs=[
        pl.BlockSpec((_BM, CROSSP), lambda i: (i, 0)),
        pl.BlockSpec((_BM,), lambda i: (i,)),
        pl.BlockSpec((CROSSP, 64), lambda i: (0, 0)),
        pl.BlockSpec((1, 64), lambda i: (0, 0)),
        pl.BlockSpec((64, 32), lambda i: (0, 0)),
        pl.BlockSpec((1, 32), lambda i: (0, 0)),
        pl.BlockSpec((1, 32), lambda i: (0, 0)),
        pl.BlockSpec(memory_space=pltpu.SMEM),
    ],
    out_specs=pl.BlockSpec((_BM,), lambda i: (i,)),
    out_shape=jax.ShapeDtypeStruct((B,), jnp.float32),
)


_iu_pad = np.concatenate([_IU, _IU[:PPAD - P]])
_ju_pad = np.concatenate([_JU, _JU[:PPAD - P]])
_SEL_A = np.zeros((PPAD, F), dtype=np.float32)
_SEL_A[np.arange(PPAD), _iu_pad] = 1.0
_SEL_B = np.zeros((PPAD, F), dtype=np.float32)
_SEL_B[np.arange(PPAD), _ju_pad] = 1.0
_CA = ((_ju_pad.astype(np.int64) * V) + _iu_pad * FD).astype(np.int32)
_CB = ((_iu_pad.astype(np.int64) * V) + _ju_pad * FD).astype(np.int32)


def kernel(x, ffm_emb, lin_emb, lin_bias, W1, b1, W2, b2, W3, b3):
    x32 = x.astype(jnp.int32)
    xo = x32 + jnp.asarray(_OFF)[None, :]
    xtf = x32.T.astype(jnp.float32)
    idxa = (jnp.dot(jnp.asarray(_SEL_A), xtf,
                    preferred_element_type=jnp.float32).astype(jnp.int32)
            + jnp.asarray(_CA)[:, None]).reshape(PPAD, NW, 128)
    idxb = (jnp.dot(jnp.asarray(_SEL_B), xtf,
                    preferred_element_type=jnp.float32).astype(jnp.int32)
            + jnp.asarray(_CB)[:, None]).reshape(PPAD, NW, 128)
    lidx = xo.T.reshape(F, NW, ROWS_PER_W)
    ffm_flat = ffm_emb.reshape(F * V, D)
    lin_flat = lin_emb.reshape(V)

    cross, lin = _make_sc_gather_cross()(ffm_flat, idxa, idxb, lidx, lin_flat)

    w1p = jnp.pad(W1, ((0, CROSSP - P * D), (0, 0)))
    cb = (lin_bias + b3).reshape(1).astype(jnp.float32)
    return _tc_mlp(cross, lin, w1p, b1.reshape(1, 64), W2, b2.reshape(1, 32),
                   W3.reshape(1, 32), cb)
